# baseline (speedup 1.0000x reference)
import functools

import jax
import jax.numpy as jnp
from jax import lax
from jax.experimental import pallas as pl
from jax.experimental.pallas import tpu as pltpu
from jax.experimental.pallas import tpu_sc as plsc

VOCAB = 50257
MAX_POS = 2048
D = 1024
B = 4
S = 2048

NC = 2
NS = 16
NW = NC * NS
PER_W = (B * S) // NW
K = 32
NCH = PER_W // K
LANES = 16
CPR = D // LANES

_mesh = plsc.VectorSubcoreMesh(core_axis_name="c", subcore_axis_name="s")


@functools.partial(
    pl.kernel,
    mesh=_mesh,
    out_type=jax.ShapeDtypeStruct((B * S, D), jnp.float32),
    scratch_types=[
        pltpu.VMEM((NCH, K), jnp.int32),
        pltpu.VMEM((K, D), jnp.float32),
        pltpu.VMEM((K, D), jnp.float32),
        pltpu.SemaphoreType.DMA,
    ],
)
def _emb_kernel(ids_hbm, wte_hbm, wpe_hbm, out_hbm, idx_v, gbuf, pbuf, sem):
    cid = lax.axis_index("c")
    sid = lax.axis_index("s")
    wid = sid * NC + cid
    base = wid * PER_W

    pltpu.sync_copy(ids_hbm.at[wid], idx_v)

    for j in range(NCH):
        row0 = pl.multiple_of(base + j * K, K)
        pos0 = pl.multiple_of(lax.bitwise_and(row0, S - 1), K)
        gather = pltpu.async_copy(wte_hbm.at[idx_v.at[j]], gbuf, sem)
        pltpu.sync_copy(wpe_hbm.at[pl.ds(pos0, K)], pbuf)
        gather.wait()

        def add_body(i, _):
            r = i // CPR
            c = (i - r * CPR) * LANES
            gbuf[r, pl.ds(c, LANES)] = (
                gbuf[r, pl.ds(c, LANES)] + pbuf[r, pl.ds(c, LANES)]
            )
            return 0

        lax.fori_loop(0, K * CPR, add_body, 0)
        pltpu.sync_copy(gbuf, out_hbm.at[pl.ds(row0, K)])


def kernel(input_ids, wte, wpe):
    ids = input_ids.reshape(NW, NCH, K).astype(jnp.int32)
    out = _emb_kernel(ids, wte, wpe)
    return out.reshape(B, S, D)


# speedup vs baseline: 1.3566x; 1.3566x over previous
import functools

import jax
import jax.numpy as jnp
from jax import lax
from jax.experimental import pallas as pl
from jax.experimental.pallas import tpu as pltpu
from jax.experimental.pallas import tpu_sc as plsc

VOCAB = 50257
MAX_POS = 2048
D = 1024
B = 4
S = 2048

NC = 2
NS = 16
NW = NC * NS
PPW = S // NW
K = 32
NH = PPW // K
NCH = NH * B
LANES = 16
CPR = D // LANES

_mesh = plsc.VectorSubcoreMesh(core_axis_name="c", subcore_axis_name="s")


@functools.partial(
    pl.kernel,
    mesh=_mesh,
    out_type=jax.ShapeDtypeStruct((B * S, D), jnp.float32),
    scratch_types=[
        pltpu.VMEM((NCH, K), jnp.int32),
        pltpu.VMEM((K, D), jnp.float32),
        pltpu.VMEM((K, D), jnp.float32),
        pltpu.VMEM((K, D), jnp.float32),
        pltpu.SemaphoreType.DMA,
        pltpu.SemaphoreType.DMA,
        pltpu.SemaphoreType.DMA,
        pltpu.SemaphoreType.DMA,
    ],
)
def _emb_kernel(ids_hbm, wte_hbm, wpe_hbm, out_hbm,
                idx_v, pbuf, gbuf0, gbuf1, gs0, gs1, ws0, ws1):
    cid = lax.axis_index("c")
    sid = lax.axis_index("s")
    wid = sid * NC + cid
    pos_base = pl.multiple_of(wid * PPW, PPW)

    pltpu.sync_copy(ids_hbm.at[wid], idx_v)

    gb = (gbuf0, gbuf1)
    gs = (gs0, gs1)
    ws = (ws0, ws1)
    write_h = [None, None]

    def out_row0(j):
        h, b = divmod(j, B)
        return pl.multiple_of(b * S + pos_base + h * K, K)

    gather_h = [None, None]
    gather_h[0] = pltpu.async_copy(wte_hbm.at[idx_v.at[0]], gb[0], gs[0])

    for j in range(NCH):
        cur = j % 2
        nxt = 1 - cur
        if j % B == 0:
            h = j // B
            pltpu.sync_copy(
                wpe_hbm.at[pl.ds(pos_base + h * K, K)], pbuf)
        if j + 1 < NCH:
            if write_h[nxt] is not None:
                write_h[nxt].wait()
            gather_h[nxt] = pltpu.async_copy(
                wte_hbm.at[idx_v.at[j + 1]], gb[nxt], gs[nxt])
        gather_h[cur].wait()

        g = gb[cur]

        def add_row(r, _):
            for c in range(CPR):
                v = pbuf[r, pl.ds(c * LANES, LANES)]
                plsc.addupdate(g.at[r, pl.ds(c * LANES, LANES)], v)
            return 0

        lax.fori_loop(0, K, add_row, 0)
        write_h[cur] = pltpu.async_copy(
            g, out_hbm.at[pl.ds(out_row0(j), K)], ws[cur])

    write_h[0].wait()
    write_h[1].wait()


def kernel(input_ids, wte, wpe):
    ids = input_ids.astype(jnp.int32).reshape(B, NW, NH, K)
    ids = jnp.transpose(ids, (1, 2, 0, 3)).reshape(NW, NCH, K)
    out = _emb_kernel(ids, wte, wpe)
    return out.reshape(B, S, D)


# speedup vs baseline: 1.9376x; 1.4283x over previous
import functools

import jax
import jax.numpy as jnp
from jax import lax
from jax.experimental import pallas as pl
from jax.experimental.pallas import tpu as pltpu
from jax.experimental.pallas import tpu_sc as plsc

VOCAB = 50257
MAX_POS = 2048
D = 1024
B = 4
S = 2048

NC = 2
NS = 16
NW = NC * NS
PPW = S // NW
K = 32
NH = PPW // K
NCH = NH * B
LANES = 16
CPR = D // LANES

_mesh = plsc.VectorSubcoreMesh(core_axis_name="c", subcore_axis_name="s")


@functools.partial(
    pl.kernel,
    mesh=_mesh,
    out_type=jax.ShapeDtypeStruct((B * S, D), jnp.float32),
    scratch_types=[
        pltpu.VMEM((NCH, K), jnp.int32),
        pltpu.VMEM((K, D), jnp.float32),
        pltpu.VMEM((K, D), jnp.float32),
        pltpu.VMEM((K, D), jnp.float32),
        pltpu.SemaphoreType.DMA,
        pltpu.SemaphoreType.DMA,
        pltpu.SemaphoreType.DMA,
        pltpu.SemaphoreType.DMA,
    ],
)
def _emb_kernel(ids_hbm, wte_hbm, wpe_hbm, out_hbm,
                idx_v, pbuf, gbuf0, gbuf1, gs0, gs1, ws0, ws1):
    cid = lax.axis_index("c")
    sid = lax.axis_index("s")
    wid = sid * NC + cid
    pos_base = pl.multiple_of(wid * PPW, PPW)

    pltpu.sync_copy(ids_hbm.at[wid], idx_v)

    gb = (gbuf0, gbuf1)
    gs = (gs0, gs1)
    ws = (ws0, ws1)
    write_h = [None, None]

    def out_row0(j):
        h, b = divmod(j, B)
        return pl.multiple_of(b * S + pos_base + h * K, K)

    gather_h = [None, None]
    gather_h[0] = pltpu.async_copy(wte_hbm.at[idx_v.at[0]], gb[0], gs[0])

    for j in range(NCH):
        cur = j % 2
        nxt = 1 - cur
        if j % B == 0:
            h = j // B
            pltpu.sync_copy(
                wpe_hbm.at[pl.ds(pos_base + h * K, K)], pbuf)
        if j + 1 < NCH:
            if write_h[nxt] is not None:
                write_h[nxt].wait()
            gather_h[nxt] = pltpu.async_copy(
                wte_hbm.at[idx_v.at[j + 1]], gb[nxt], gs[nxt])
        gather_h[cur].wait()

        g = gb[cur]

        @plsc.parallel_loop(0, K, unroll=2)
        def add_row(r):
            for c in range(CPR):
                v = pbuf[r, pl.ds(c * LANES, LANES)]
                plsc.addupdate(g.at[r, pl.ds(c * LANES, LANES)], v)
        write_h[cur] = pltpu.async_copy(
            g, out_hbm.at[pl.ds(out_row0(j), K)], ws[cur])

    write_h[0].wait()
    write_h[1].wait()


def kernel(input_ids, wte, wpe):
    ids = input_ids.astype(jnp.int32).reshape(B, NW, NH, K)
    ids = jnp.transpose(ids, (1, 2, 0, 3)).reshape(NW, NCH, K)
    out = _emb_kernel(ids, wte, wpe)
    return out.reshape(B, S, D)
